# Initial kernel scaffold; baseline (speedup 1.0000x reference)
import jax
import jax.numpy as jnp
from jax.experimental import pallas as pl


def kernel(x, edge_index, W, b):
    raise NotImplementedError("write your pallas kernel here")


# speedup vs baseline: 1.8307x; 1.8307x over previous
import jax
import jax.numpy as jnp
from jax import lax
from jax.experimental import pallas as pl
from jax.experimental.pallas import tpu as pltpu
from jax.experimental.pallas import tpu_sc as plsc

N_NODES = 10000
N_FEAT = 256
N_EDGES = 160000

NW = 32
ROWS_PER_W = 320
N_PAD = NW * ROWS_PER_W

ECHUNK = 3200
NCHUNK = N_EDGES // ECHUNK
GB = 64

LANES = 16
FCH = N_FEAT // LANES

NEG_INF = float("-inf")


def _scatter_max_body(x_hbm, src_hbm, dst_hbm, out_hbm,
                      acc, srcb, dstb, srcc, dlc, rows, sem):
    wid = lax.axis_index("s") * 2 + lax.axis_index("c")
    lo = wid * ROWS_PER_W
    lane = lax.iota(jnp.int32, LANES)

    def _init(i, _):
        acc[pl.ds(i * LANES, LANES)] = jnp.full((LANES,), NEG_INF, jnp.float32)
        return 0
    lax.fori_loop(0, ROWS_PER_W * FCH, _init, 0)

    def _zidx(i, _):
        srcc[pl.ds(i * LANES, LANES)] = jnp.zeros((LANES,), jnp.int32)
        return 0
    lax.fori_loop(0, (ECHUNK + 2 * LANES) // LANES, _zidx, 0)

    def _chunk(c, _):
        e0 = c * ECHUNK
        pltpu.sync_copy(src_hbm.at[pl.ds(e0, ECHUNK)], srcb)
        pltpu.sync_copy(dst_hbm.at[pl.ds(e0, ECHUNK)], dstb)

        def _filt(v, n_vec):
            d16 = dstb[pl.ds(v * LANES, LANES)]
            s16 = srcb[pl.ds(v * LANES, LANES)]
            dl = d16 - lo
            msk = (dl >= 0) & (dl < ROWS_PER_W)
            cs = plsc.cumsum(msk.astype(jnp.int32))
            pos = jnp.where(msk, n_vec + cs - 1, ECHUNK + LANES)
            plsc.store_scatter(srcc, [pos], s16)
            plsc.store_scatter(dlc, [pos], dl)
            return n_vec + plsc.all_reduce_population_count(msk)
        n_vec = lax.fori_loop(0, ECHUNK // LANES, _filt,
                              jnp.zeros((LANES,), jnp.int32))
        n = jnp.max(n_vec)

        def _batch(bi, _):
            g = bi * GB
            nb = jnp.minimum(n - g, GB)
            pltpu.async_copy(x_hbm.at[srcc.at[pl.ds(g, GB)]], rows, sem).wait()

            def _edge(k, _):
                dlv = plsc.load_gather(dlc, [jnp.full((LANES,), g + k, jnp.int32)])
                base = dlv * N_FEAT + lane
                for j in range(FCH):
                    idx = base + (j * LANES)
                    a = plsc.load_gather(acc, [idx])
                    r = rows[k, pl.ds(j * LANES, LANES)]
                    plsc.store_scatter(acc, [idx], jnp.maximum(a, r))
                return 0
            lax.fori_loop(0, nb, _edge, 0)
            return 0
        lax.fori_loop(0, (n + GB - 1) // GB, _batch, 0)
        return 0
    lax.fori_loop(0, NCHUNK, _chunk, 0)

    pltpu.sync_copy(acc, out_hbm.at[pl.ds(wid * ROWS_PER_W * N_FEAT,
                                          ROWS_PER_W * N_FEAT)])


def _segment_max_rows(x, src, dst):
    mesh = plsc.VectorSubcoreMesh(core_axis_name="c", subcore_axis_name="s")
    kfn = pl.kernel(
        _scatter_max_body,
        out_type=jax.ShapeDtypeStruct((N_PAD * N_FEAT,), jnp.float32),
        mesh=mesh,
        compiler_params=pltpu.CompilerParams(needs_layout_passes=False),
        scratch_types=[
            pltpu.VMEM((ROWS_PER_W * N_FEAT,), jnp.float32),
            pltpu.VMEM((ECHUNK,), jnp.int32),
            pltpu.VMEM((ECHUNK,), jnp.int32),
            pltpu.VMEM((ECHUNK + 2 * LANES,), jnp.int32),
            pltpu.VMEM((ECHUNK + 2 * LANES,), jnp.int32),
            pltpu.VMEM((GB, N_FEAT), jnp.float32),
            pltpu.SemaphoreType.DMA,
        ],
    )
    return kfn(x, src, dst)


def _mlp_body(x_ref, m_ref, wt_ref, b_ref, o_ref):
    xb = x_ref[...]
    xj = m_ref[...] - xb
    xj = jnp.where(xj < -10000.0, 0.0, xj)
    acc = jnp.dot(xb, wt_ref[0:N_FEAT, :], preferred_element_type=jnp.float32)
    acc += jnp.dot(xj, wt_ref[N_FEAT:2 * N_FEAT, :],
                   preferred_element_type=jnp.float32)
    o_ref[...] = jnp.maximum(acc + b_ref[...], 0.0)


def _mlp(x, m, Wt, b2):
    blk = 2000
    grid = N_NODES // blk
    return pl.pallas_call(
        _mlp_body,
        grid=(grid,),
        in_specs=[
            pl.BlockSpec((blk, N_FEAT), lambda i: (i, 0)),
            pl.BlockSpec((blk, N_FEAT), lambda i: (i, 0)),
            pl.BlockSpec((2 * N_FEAT, N_FEAT), lambda i: (0, 0)),
            pl.BlockSpec((1, N_FEAT), lambda i: (0, 0)),
        ],
        out_specs=pl.BlockSpec((blk, N_FEAT), lambda i: (i, 0)),
        out_shape=jax.ShapeDtypeStruct((N_NODES, N_FEAT), jnp.float32),
    )(x, m, Wt, b2)


@jax.jit
def kernel(x, edge_index, W, b):
    src = edge_index[0].astype(jnp.int32)
    dst = edge_index[1].astype(jnp.int32)
    m_flat = _segment_max_rows(x, src, dst)
    m = m_flat.reshape(N_PAD, N_FEAT)[:N_NODES]
    return _mlp(x, m, W.T, b.reshape(1, N_FEAT))


# speedup vs baseline: 2.4706x; 1.3496x over previous
import jax
import jax.numpy as jnp
from jax import lax
from jax.experimental import pallas as pl
from jax.experimental.pallas import tpu as pltpu
from jax.experimental.pallas import tpu_sc as plsc

N_NODES = 10000
N_FEAT = 256
N_EDGES = 160000

NW = 32
ROWS_PER_W = 320
N_PAD = NW * ROWS_PER_W
DL_BITS = 9
DL_MASK = (1 << DL_BITS) - 1

ECHUNK = 3200
NCHUNK = N_EDGES // ECHUNK
GB = 64
LIST = 8208
FLUSH = LIST - ECHUNK

LANES = 16
FCH = N_FEAT // LANES

NEG_INF = float("-inf")


def _scatter_max_body(x_hbm, src_hbm, dst_hbm, out_hbm,
                      acc, srcb, dstb, packed, stage0, stage1,
                      rows0, rows1, sem_s, sem_d, sem0, sem1):
    wid = lax.axis_index("s") * 2 + lax.axis_index("c")
    lo = wid * ROWS_PER_W
    lane = lax.iota(jnp.int32, LANES)

    def _init(i, _):
        acc[pl.ds(i * LANES, LANES)] = jnp.full((LANES,), NEG_INF, jnp.float32)
        return 0
    lax.fori_loop(0, ROWS_PER_W * FCH, _init, 0)

    def _zidx(i, _):
        packed[pl.ds(i * LANES, LANES)] = jnp.zeros((LANES,), jnp.int32)
        return 0
    lax.fori_loop(0, (LIST + LANES) // LANES, _zidx, 0)

    def _unpack_start(bi, stage, rowsbuf, sem):
        g = bi * GB
        for q in range(GB // LANES):
            v = packed[pl.ds(g + q * LANES, LANES)]
            stage[pl.ds(q * LANES, LANES)] = lax.shift_right_logical(v, DL_BITS)
        pltpu.async_copy(x_hbm.at[stage], rowsbuf, sem)

    def _accum(bi, n, rowsbuf):
        g = bi * GB
        nb = jnp.minimum(n - g, GB)

        def _edge(k, _):
            pv = plsc.load_gather(packed, [jnp.full((LANES,), g + k, jnp.int32)])
            base = (pv & DL_MASK) * N_FEAT + lane
            for j in range(FCH):
                idx = base + (j * LANES)
                a = plsc.load_gather(acc, [idx])
                r = rowsbuf[k, pl.ds(j * LANES, LANES)]
                plsc.store_scatter(acc, [idx], jnp.maximum(a, r))
            return 0
        lax.fori_loop(0, nb, _edge, 0)

    def _flush(n):
        nbat = (n + GB - 1) // GB

        @pl.when(nbat > 0)
        def _():
            _unpack_start(0, stage0, rows0, sem0)

        def _pair(p, _):
            bp = 2 * p

            @pl.when(bp + 1 < nbat)
            def _():
                _unpack_start(bp + 1, stage1, rows1, sem1)
            pltpu.make_async_copy(x_hbm.at[stage0], rows0, sem0).wait()
            _accum(bp, n, rows0)

            @pl.when(bp + 2 < nbat)
            def _():
                _unpack_start(bp + 2, stage0, rows0, sem0)

            @pl.when(bp + 1 < nbat)
            def _():
                pltpu.make_async_copy(x_hbm.at[stage1], rows1, sem1).wait()
                _accum(bp + 1, n, rows1)
            return 0
        lax.fori_loop(0, (nbat + 1) // 2, _pair, 0)

    def _chunk(c, n_vec):
        e0 = c * ECHUNK
        cp_s = pltpu.make_async_copy(src_hbm.at[pl.ds(e0, ECHUNK)], srcb, sem_s)
        cp_d = pltpu.make_async_copy(dst_hbm.at[pl.ds(e0, ECHUNK)], dstb, sem_d)
        cp_s.start()
        cp_d.start()

        def _do_flush(nv):
            _flush(jnp.max(nv))
            return jnp.zeros((LANES,), jnp.int32)
        n_vec = lax.cond(jnp.max(n_vec) >= FLUSH, _do_flush,
                         lambda nv: nv, n_vec)
        cp_s.wait()
        cp_d.wait()

        def _filt(v, nv):
            for t in range(2):
                off = (2 * v + t) * LANES
                d16 = dstb[pl.ds(off, LANES)]
                s16 = srcb[pl.ds(off, LANES)]
                dl = d16 - lo
                msk = (dl >= 0) & (dl < ROWS_PER_W)
                cs = plsc.cumsum(msk.astype(jnp.int32))
                pos = jnp.where(msk, nv + cs - 1, LIST)
                plsc.store_scatter(
                    packed, [pos],
                    lax.shift_left(s16, DL_BITS) | (dl & DL_MASK))
                nv = nv + plsc.all_reduce_population_count(msk)
            return nv
        return lax.fori_loop(0, ECHUNK // (2 * LANES), _filt, n_vec)

    n_vec = lax.fori_loop(0, NCHUNK, _chunk, jnp.zeros((LANES,), jnp.int32))
    _flush(jnp.max(n_vec))

    pltpu.sync_copy(acc, out_hbm.at[pl.ds(wid * ROWS_PER_W * N_FEAT,
                                          ROWS_PER_W * N_FEAT)])


def _segment_max_rows(x, src, dst):
    mesh = plsc.VectorSubcoreMesh(core_axis_name="c", subcore_axis_name="s")
    kfn = pl.kernel(
        _scatter_max_body,
        out_type=jax.ShapeDtypeStruct((N_PAD * N_FEAT,), jnp.float32),
        mesh=mesh,
        compiler_params=pltpu.CompilerParams(needs_layout_passes=False),
        scratch_types=[
            pltpu.VMEM((ROWS_PER_W * N_FEAT,), jnp.float32),
            pltpu.VMEM((ECHUNK,), jnp.int32),
            pltpu.VMEM((ECHUNK,), jnp.int32),
            pltpu.VMEM((LIST + LANES,), jnp.int32),
            pltpu.VMEM((GB,), jnp.int32),
            pltpu.VMEM((GB,), jnp.int32),
            pltpu.VMEM((GB, N_FEAT), jnp.float32),
            pltpu.VMEM((GB, N_FEAT), jnp.float32),
            pltpu.SemaphoreType.DMA,
            pltpu.SemaphoreType.DMA,
            pltpu.SemaphoreType.DMA,
            pltpu.SemaphoreType.DMA,
        ],
    )
    return kfn(x, src, dst)


def _mlp_body(x_ref, m_ref, wt_ref, b_ref, o_ref):
    xb = x_ref[...]
    xj = m_ref[...] - xb
    xj = jnp.where(xj < -10000.0, 0.0, xj)
    acc = jnp.dot(xb, wt_ref[0:N_FEAT, :], preferred_element_type=jnp.float32)
    acc += jnp.dot(xj, wt_ref[N_FEAT:2 * N_FEAT, :],
                   preferred_element_type=jnp.float32)
    o_ref[...] = jnp.maximum(acc + b_ref[...], 0.0)


def _mlp(x, m, Wt, b2):
    blk = 2000
    grid = N_NODES // blk
    return pl.pallas_call(
        _mlp_body,
        grid=(grid,),
        in_specs=[
            pl.BlockSpec((blk, N_FEAT), lambda i: (i, 0)),
            pl.BlockSpec((blk, N_FEAT), lambda i: (i, 0)),
            pl.BlockSpec((2 * N_FEAT, N_FEAT), lambda i: (0, 0)),
            pl.BlockSpec((1, N_FEAT), lambda i: (0, 0)),
        ],
        out_specs=pl.BlockSpec((blk, N_FEAT), lambda i: (i, 0)),
        out_shape=jax.ShapeDtypeStruct((N_NODES, N_FEAT), jnp.float32),
    )(x, m, Wt, b2)


@jax.jit
def kernel(x, edge_index, W, b):
    src = edge_index[0].astype(jnp.int32)
    dst = edge_index[1].astype(jnp.int32)
    m_flat = _segment_max_rows(x, src, dst)
    m = m_flat.reshape(N_PAD, N_FEAT)[:N_NODES]
    return _mlp(x, m, W.T, b.reshape(1, N_FEAT))


# speedup vs baseline: 2.7758x; 1.1235x over previous
import jax
import jax.numpy as jnp
from jax import lax
from jax.experimental import pallas as pl
from jax.experimental.pallas import tpu as pltpu
from jax.experimental.pallas import tpu_sc as plsc

N_NODES = 10000
N_FEAT = 256
N_EDGES = 160000

NW = 32
ROWS_PER_W = 320
N_PAD = NW * ROWS_PER_W
DL_BITS = 9
DL_MASK = (1 << DL_BITS) - 1

ECHUNK = 2000
NCHUNK = N_EDGES // ECHUNK
GB = 32
LIST = 8208
FLUSH = LIST - ECHUNK

LANES = 16
FCH = N_FEAT // LANES
FILT_UNROLL = 5
EDGE_UNROLL = 4

NEG_INF = float("-inf")


def _scatter_max_body(x_hbm, src_hbm, dst_hbm, out_hbm,
                      acc, srcb0, dstb0, srcb1, dstb1, packed,
                      stage0, stage1, rows0, rows1,
                      sem_s0, sem_d0, sem_s1, sem_d1, sem0, sem1):
    wid = lax.axis_index("s") * 2 + lax.axis_index("c")
    lo = wid * ROWS_PER_W
    lane = lax.iota(jnp.int32, LANES)

    def _init(i, _):
        acc[pl.ds(i * LANES, LANES)] = jnp.full((LANES,), NEG_INF, jnp.float32)
        return 0
    lax.fori_loop(0, ROWS_PER_W * FCH, _init, 0)

    def _zidx(i, _):
        packed[pl.ds(i * LANES, LANES)] = jnp.zeros((LANES,), jnp.int32)
        return 0
    lax.fori_loop(0, (LIST + LANES) // LANES, _zidx, 0)

    def _unpack_start(bi, stage, rowsbuf, sem):
        g = bi * GB
        for q in range(GB // LANES):
            v = packed[pl.ds(g + q * LANES, LANES)]
            stage[pl.ds(q * LANES, LANES)] = jnp.right_shift(v, DL_BITS)
        pltpu.async_copy(x_hbm.at[stage], rowsbuf, sem)

    def _edge_body(i, rowsbuf, g):
        pkv = packed[pl.ds(g + i, LANES)]
        base = (pkv[0] & DL_MASK) * N_FEAT
        for j in range(FCH):
            a = acc[pl.ds(base + j * LANES, LANES)]
            r = rowsbuf[i, pl.ds(j * LANES, LANES)]
            acc[pl.ds(base + j * LANES, LANES)] = jnp.maximum(a, r)

    def _accum_full(bi, rowsbuf):
        g = bi * GB

        def _edges(k, _):
            for u in range(EDGE_UNROLL):
                _edge_body(k * EDGE_UNROLL + u, rowsbuf, g)
            return 0
        lax.fori_loop(0, GB // EDGE_UNROLL, _edges, 0)

    def _accum_tail(bi, n, rowsbuf):
        g = bi * GB
        nb = jnp.minimum(n - g, GB)

        def _edges(k, _):
            _edge_body(k, rowsbuf, g)
            return 0
        lax.fori_loop(0, nb, _edges, 0)

    def _accum(bi, n, rowsbuf):
        full = (bi + 1) * GB <= n

        @pl.when(full)
        def _():
            _accum_full(bi, rowsbuf)

        @pl.when(jnp.logical_not(full))
        def _():
            _accum_tail(bi, n, rowsbuf)

    def _flush(n):
        nbat = (n + GB - 1) // GB

        @pl.when(nbat > 0)
        def _():
            _unpack_start(0, stage0, rows0, sem0)

        def _pair(p, _):
            bp = 2 * p

            @pl.when(bp + 1 < nbat)
            def _():
                _unpack_start(bp + 1, stage1, rows1, sem1)
            pltpu.make_async_copy(x_hbm.at[stage0], rows0, sem0).wait()
            _accum(bp, n, rows0)

            @pl.when(bp + 2 < nbat)
            def _():
                _unpack_start(bp + 2, stage0, rows0, sem0)

            @pl.when(bp + 1 < nbat)
            def _():
                pltpu.make_async_copy(x_hbm.at[stage1], rows1, sem1).wait()
                _accum(bp + 1, n, rows1)
            return 0
        lax.fori_loop(0, (nbat + 1) // 2, _pair, 0)

    def _start_stage(c, srcb, dstb, sem_s, sem_d):
        e0 = c * ECHUNK
        pltpu.make_async_copy(
            src_hbm.at[pl.ds(e0, ECHUNK)], srcb, sem_s).start()
        pltpu.make_async_copy(
            dst_hbm.at[pl.ds(e0, ECHUNK)], dstb, sem_d).start()

    def _wait_stage(c, srcb, dstb, sem_s, sem_d):
        e0 = c * ECHUNK
        pltpu.make_async_copy(
            src_hbm.at[pl.ds(e0, ECHUNK)], srcb, sem_s).wait()
        pltpu.make_async_copy(
            dst_hbm.at[pl.ds(e0, ECHUNK)], dstb, sem_d).wait()

    def _maybe_flush(n_vec):
        def _do(nv):
            _flush(jnp.max(nv))
            return jnp.zeros((LANES,), jnp.int32)
        return lax.cond(jnp.max(n_vec) >= FLUSH, _do, lambda nv: nv, n_vec)

    def _filter(srcb, dstb, n_vec):
        def _filt(v, nv):
            for t in range(FILT_UNROLL):
                off = (FILT_UNROLL * v + t) * LANES
                d16 = dstb[pl.ds(off, LANES)]
                s16 = srcb[pl.ds(off, LANES)]
                dl = d16 - lo
                msk = (dl >= 0) & (dl < ROWS_PER_W)
                cs = plsc.cumsum(msk.astype(jnp.int32))
                pos = jnp.where(msk, nv + cs - 1, LIST)
                plsc.store_scatter(
                    packed, [pos],
                    jnp.left_shift(s16, DL_BITS) | (dl & DL_MASK))
                nv = nv + plsc.all_reduce_population_count(msk)
            return nv
        return lax.fori_loop(0, ECHUNK // (FILT_UNROLL * LANES), _filt, n_vec)

    _start_stage(0, srcb0, dstb0, sem_s0, sem_d0)

    def _cpair(p, n_vec):
        c0 = 2 * p
        _start_stage(c0 + 1, srcb1, dstb1, sem_s1, sem_d1)
        n_vec = _maybe_flush(n_vec)
        _wait_stage(c0, srcb0, dstb0, sem_s0, sem_d0)
        n_vec = _filter(srcb0, dstb0, n_vec)

        @pl.when(c0 + 2 < NCHUNK)
        def _():
            _start_stage(c0 + 2, srcb0, dstb0, sem_s0, sem_d0)
        n_vec = _maybe_flush(n_vec)
        _wait_stage(c0 + 1, srcb1, dstb1, sem_s1, sem_d1)
        n_vec = _filter(srcb1, dstb1, n_vec)
        return n_vec

    n_vec = lax.fori_loop(0, NCHUNK // 2, _cpair,
                          jnp.zeros((LANES,), jnp.int32))
    _flush(jnp.max(n_vec))

    pltpu.sync_copy(acc, out_hbm.at[pl.ds(wid * ROWS_PER_W * N_FEAT,
                                          ROWS_PER_W * N_FEAT)])


def _segment_max_rows(x, src, dst):
    mesh = plsc.VectorSubcoreMesh(core_axis_name="c", subcore_axis_name="s")
    kfn = pl.kernel(
        _scatter_max_body,
        out_type=jax.ShapeDtypeStruct((N_PAD * N_FEAT,), jnp.float32),
        mesh=mesh,
        compiler_params=pltpu.CompilerParams(needs_layout_passes=False),
        scratch_types=[
            pltpu.VMEM((ROWS_PER_W * N_FEAT,), jnp.float32),
            pltpu.VMEM((ECHUNK,), jnp.int32),
            pltpu.VMEM((ECHUNK,), jnp.int32),
            pltpu.VMEM((ECHUNK,), jnp.int32),
            pltpu.VMEM((ECHUNK,), jnp.int32),
            pltpu.VMEM((LIST + LANES,), jnp.int32),
            pltpu.VMEM((GB,), jnp.int32),
            pltpu.VMEM((GB,), jnp.int32),
            pltpu.VMEM((GB, N_FEAT), jnp.float32),
            pltpu.VMEM((GB, N_FEAT), jnp.float32),
            pltpu.SemaphoreType.DMA,
            pltpu.SemaphoreType.DMA,
            pltpu.SemaphoreType.DMA,
            pltpu.SemaphoreType.DMA,
            pltpu.SemaphoreType.DMA,
            pltpu.SemaphoreType.DMA,
        ],
    )
    return kfn(x, src, dst)


def _mlp_body(x_ref, m_ref, wt_ref, b_ref, o_ref):
    xb = x_ref[...]
    xj = m_ref[...] - xb
    xj = jnp.where(xj < -10000.0, 0.0, xj)
    acc = jnp.dot(xb, wt_ref[0:N_FEAT, :], preferred_element_type=jnp.float32)
    acc += jnp.dot(xj, wt_ref[N_FEAT:2 * N_FEAT, :],
                   preferred_element_type=jnp.float32)
    o_ref[...] = jnp.maximum(acc + b_ref[...], 0.0)


def _mlp(x, m, Wt, b2):
    blk = 2000
    grid = N_NODES // blk
    return pl.pallas_call(
        _mlp_body,
        grid=(grid,),
        in_specs=[
            pl.BlockSpec((blk, N_FEAT), lambda i: (i, 0)),
            pl.BlockSpec((blk, N_FEAT), lambda i: (i, 0)),
            pl.BlockSpec((2 * N_FEAT, N_FEAT), lambda i: (0, 0)),
            pl.BlockSpec((1, N_FEAT), lambda i: (0, 0)),
        ],
        out_specs=pl.BlockSpec((blk, N_FEAT), lambda i: (i, 0)),
        out_shape=jax.ShapeDtypeStruct((N_NODES, N_FEAT), jnp.float32),
    )(x, m, Wt, b2)


@jax.jit
def kernel(x, edge_index, W, b):
    src = edge_index[0].astype(jnp.int32)
    dst = edge_index[1].astype(jnp.int32)
    m_flat = _segment_max_rows(x, src, dst)
    m = m_flat.reshape(N_PAD, N_FEAT)[:N_NODES]
    return _mlp(x, m, W.T, b.reshape(1, N_FEAT))


# speedup vs baseline: 5.4758x; 1.9727x over previous
import jax
import jax.numpy as jnp
from jax import lax
from jax.experimental import pallas as pl
from jax.experimental.pallas import tpu as pltpu
from jax.experimental.pallas import tpu_sc as plsc

N_NODES = 10000
N_FEAT = 256
N_EDGES = 160000

NW = 32
ROWS_PER_W = 320
N_PAD = NW * ROWS_PER_W
DL_BITS = 9
DL_MASK = (1 << DL_BITS) - 1

ECHUNK = 2000
NCHUNK = N_EDGES // ECHUNK
GB = 32
LIST = 8208
FLUSH = LIST - ECHUNK

LANES = 16
FCH = N_FEAT // LANES
FILT_UNROLL = 5
EDGE_UNROLL = 16

NEG_INF = float("-inf")


def _scatter_max_body(x_hbm, src_hbm, dst_hbm, out_hbm,
                      acc, srcb0, dstb0, srcb1, dstb1, packed,
                      stage0, stage1, rows0, rows1,
                      sem_s0, sem_d0, sem_s1, sem_d1, sem0, sem1):
    wid = lax.axis_index("s") * 2 + lax.axis_index("c")
    lo = wid * ROWS_PER_W
    lane = lax.iota(jnp.int32, LANES)

    ninf = jnp.full((LANES,), NEG_INF, jnp.float32)

    def _init(r, _):
        for j in range(FCH):
            acc[r, pl.ds(j * LANES, LANES)] = ninf
        return 0
    lax.fori_loop(0, ROWS_PER_W, _init, 0)

    def _zidx(i, _):
        packed[pl.ds(i * LANES, LANES)] = jnp.zeros((LANES,), jnp.int32)
        return 0
    lax.fori_loop(0, (LIST + LANES) // LANES, _zidx, 0)

    def _unpack_start(bi, stage, rowsbuf, sem):
        g = bi * GB
        for q in range(GB // LANES):
            v = packed[pl.ds(g + q * LANES, LANES)]
            stage[pl.ds(q * LANES, LANES)] = jnp.right_shift(v, DL_BITS)
        pltpu.async_copy(x_hbm.at[stage], rowsbuf, sem)

    def _edge_rows(i, rowsbuf, row):
        rvs = [rowsbuf[i, pl.ds(j * LANES, LANES)] for j in range(FCH)]
        avs = [acc[row, pl.ds(j * LANES, LANES)] for j in range(FCH)]
        for j in range(FCH):
            acc[row, pl.ds(j * LANES, LANES)] = jnp.maximum(avs[j], rvs[j])

    def _accum_full(bi, rowsbuf):
        g = bi * GB

        def _edges(k, _):
            pkv = packed[pl.ds(g + k * EDGE_UNROLL, LANES)]
            rows = [pkv[u] & DL_MASK for u in range(EDGE_UNROLL)]
            for u in range(EDGE_UNROLL):
                _edge_rows(k * EDGE_UNROLL + u, rowsbuf, rows[u])
            return 0
        lax.fori_loop(0, GB // EDGE_UNROLL, _edges, 0)

    def _accum_tail(bi, n, rowsbuf):
        g = bi * GB
        nb = jnp.minimum(n - g, GB)

        def _edges(k, _):
            pkv = packed[pl.ds(g + k, LANES)]
            _edge_rows(k, rowsbuf, pkv[0] & DL_MASK)
            return 0
        lax.fori_loop(0, nb, _edges, 0)

    def _accum(bi, n, rowsbuf):
        full = (bi + 1) * GB <= n

        @pl.when(full)
        def _():
            _accum_full(bi, rowsbuf)

        @pl.when(jnp.logical_not(full))
        def _():
            _accum_tail(bi, n, rowsbuf)

    def _flush(n):
        nbat = (n + GB - 1) // GB

        @pl.when(nbat > 0)
        def _():
            _unpack_start(0, stage0, rows0, sem0)

        def _pair(p, _):
            bp = 2 * p

            @pl.when(bp + 1 < nbat)
            def _():
                _unpack_start(bp + 1, stage1, rows1, sem1)
            pltpu.make_async_copy(x_hbm.at[stage0], rows0, sem0).wait()
            _accum(bp, n, rows0)

            @pl.when(bp + 2 < nbat)
            def _():
                _unpack_start(bp + 2, stage0, rows0, sem0)

            @pl.when(bp + 1 < nbat)
            def _():
                pltpu.make_async_copy(x_hbm.at[stage1], rows1, sem1).wait()
                _accum(bp + 1, n, rows1)
            return 0
        lax.fori_loop(0, (nbat + 1) // 2, _pair, 0)

    def _start_stage(c, srcb, dstb, sem_s, sem_d):
        e0 = c * ECHUNK
        pltpu.make_async_copy(
            src_hbm.at[pl.ds(e0, ECHUNK)], srcb, sem_s).start()
        pltpu.make_async_copy(
            dst_hbm.at[pl.ds(e0, ECHUNK)], dstb, sem_d).start()

    def _wait_stage(c, srcb, dstb, sem_s, sem_d):
        e0 = c * ECHUNK
        pltpu.make_async_copy(
            src_hbm.at[pl.ds(e0, ECHUNK)], srcb, sem_s).wait()
        pltpu.make_async_copy(
            dst_hbm.at[pl.ds(e0, ECHUNK)], dstb, sem_d).wait()

    def _maybe_flush(nvm1_vec):
        def _do(nv):
            _flush(jnp.max(nv) + 1)
            return jnp.full((LANES,), -1, jnp.int32)
        return lax.cond(jnp.max(nvm1_vec) >= FLUSH - 1, _do,
                        lambda nv: nv, nvm1_vec)

    def _filter(srcb, dstb, n_vec):
        def _filt(v, nvm1):
            offs = [(FILT_UNROLL * v + t) * LANES for t in range(FILT_UNROLL)]
            dv = [dstb[pl.ds(o, LANES)] for o in offs]
            sv = [srcb[pl.ds(o, LANES)] for o in offs]
            for t in range(FILT_UNROLL):
                dl = dv[t] - lo
                msk = (dl >= 0) & (dl < ROWS_PER_W)
                cs = plsc.cumsum(jnp.ones((LANES,), jnp.int32), mask=msk)
                plsc.store_scatter(
                    packed, [nvm1 + cs],
                    jnp.left_shift(sv[t], DL_BITS) | dl, mask=msk)
                nvm1 = nvm1 + plsc.all_reduce_population_count(msk)
            return nvm1
        return lax.fori_loop(0, ECHUNK // (FILT_UNROLL * LANES), _filt, n_vec)

    _start_stage(0, srcb0, dstb0, sem_s0, sem_d0)

    def _cpair(p, n_vec):
        c0 = 2 * p
        _start_stage(c0 + 1, srcb1, dstb1, sem_s1, sem_d1)
        n_vec = _maybe_flush(n_vec)
        _wait_stage(c0, srcb0, dstb0, sem_s0, sem_d0)
        n_vec = _filter(srcb0, dstb0, n_vec)

        @pl.when(c0 + 2 < NCHUNK)
        def _():
            _start_stage(c0 + 2, srcb0, dstb0, sem_s0, sem_d0)
        n_vec = _maybe_flush(n_vec)
        _wait_stage(c0 + 1, srcb1, dstb1, sem_s1, sem_d1)
        n_vec = _filter(srcb1, dstb1, n_vec)
        return n_vec

    n_vec = lax.fori_loop(0, NCHUNK // 2, _cpair,
                          jnp.full((LANES,), -1, jnp.int32))
    _flush(jnp.max(n_vec) + 1)

    pltpu.sync_copy(acc, out_hbm.at[pl.ds(wid * ROWS_PER_W, ROWS_PER_W)])


def _segment_max_rows(x, src, dst):
    mesh = plsc.VectorSubcoreMesh(core_axis_name="c", subcore_axis_name="s")
    kfn = pl.kernel(
        _scatter_max_body,
        out_type=jax.ShapeDtypeStruct((N_PAD, N_FEAT), jnp.float32),
        mesh=mesh,
        compiler_params=pltpu.CompilerParams(needs_layout_passes=False),
        scratch_types=[
            pltpu.VMEM((ROWS_PER_W, N_FEAT), jnp.float32),
            pltpu.VMEM((ECHUNK,), jnp.int32),
            pltpu.VMEM((ECHUNK,), jnp.int32),
            pltpu.VMEM((ECHUNK,), jnp.int32),
            pltpu.VMEM((ECHUNK,), jnp.int32),
            pltpu.VMEM((LIST + LANES,), jnp.int32),
            pltpu.VMEM((GB,), jnp.int32),
            pltpu.VMEM((GB,), jnp.int32),
            pltpu.VMEM((GB, N_FEAT), jnp.float32),
            pltpu.VMEM((GB, N_FEAT), jnp.float32),
            pltpu.SemaphoreType.DMA,
            pltpu.SemaphoreType.DMA,
            pltpu.SemaphoreType.DMA,
            pltpu.SemaphoreType.DMA,
            pltpu.SemaphoreType.DMA,
            pltpu.SemaphoreType.DMA,
        ],
    )
    return kfn(x, src, dst)


def _mlp_body(x_ref, m_ref, w_ref, b_ref, o_ref):
    xb = x_ref[...]
    xj = m_ref[...] - xb
    xj = jnp.where(xj < -10000.0, 0.0, xj)
    dn = (((1,), (1,)), ((), ()))
    acc = lax.dot_general(xb, w_ref[:, 0:N_FEAT], dn,
                          preferred_element_type=jnp.float32)
    acc += lax.dot_general(xj, w_ref[:, N_FEAT:2 * N_FEAT], dn,
                           preferred_element_type=jnp.float32)
    o_ref[...] = jnp.maximum(acc + b_ref[...], 0.0)


def _mlp(x, m, W, b2):
    blk = 2000
    grid = N_NODES // blk
    return pl.pallas_call(
        _mlp_body,
        grid=(grid,),
        in_specs=[
            pl.BlockSpec((blk, N_FEAT), lambda i: (i, 0)),
            pl.BlockSpec((blk, N_FEAT), lambda i: (i, 0)),
            pl.BlockSpec((N_FEAT, 2 * N_FEAT), lambda i: (0, 0)),
            pl.BlockSpec((1, N_FEAT), lambda i: (0, 0)),
        ],
        out_specs=pl.BlockSpec((blk, N_FEAT), lambda i: (i, 0)),
        out_shape=jax.ShapeDtypeStruct((N_NODES, N_FEAT), jnp.float32),
    )(x, m, W, b2)


@jax.jit
def kernel(x, edge_index, W, b):
    src = edge_index[0].astype(jnp.int32)
    dst = edge_index[1].astype(jnp.int32)
    m = _segment_max_rows(x, src, dst)
    return _mlp(x, m, W, b.reshape(1, N_FEAT))
